# baseline (speedup 1.0000x reference)
import functools

import jax
import jax.numpy as jnp
from jax import lax
from jax.experimental import pallas as pl
from jax.experimental.pallas import tpu as pltpu
from jax.experimental.pallas import tpu_sc as plsc


def _table_body(emb_ref, w_ref, b_ref, out_ref):
    acc = lax.dot_general(
        emb_ref[...], w_ref[...],
        dimension_numbers=(((1,), (1,)), ((), ())),
        preferred_element_type=jnp.float32,
    )
    out_ref[...] = acc + b_ref[...][None, :]


def _make_table(emb_table, head_w, head_b):
    V, D = emb_table.shape
    return pl.pallas_call(
        _table_body,
        out_shape=jax.ShapeDtypeStruct((V, V), jnp.float32),
    )(emb_table, head_w, head_b)


def _make_gather(N, V, n_workers, chunk):
    rows_per_w = N // n_workers
    n_chunks = rows_per_w // chunk
    mesh = plsc.VectorSubcoreMesh(core_axis_name="c", subcore_axis_name="s")

    @functools.partial(
        pl.kernel,
        mesh=mesh,
        out_type=jax.ShapeDtypeStruct((N, V), jnp.float32),
        scratch_types=[
            pltpu.VMEM((chunk,), jnp.int32),
            pltpu.VMEM((chunk, V), jnp.float32),
            pltpu.SemaphoreType.DMA,
        ],
        compiler_params=pltpu.CompilerParams(use_tc_tiling_on_sc=False),
    )
    def gather(table_hbm, idx_hbm, out_hbm, idx_v, rows_v, sem):
        n_cores = lax.axis_size("c")
        wid = lax.axis_index("s") * n_cores + lax.axis_index("c")
        base = wid * rows_per_w

        def body(i, carry):
            off = base + i * chunk
            pltpu.sync_copy(idx_hbm.at[pl.ds(off, chunk)], idx_v)
            pltpu.async_copy(table_hbm.at[idx_v], rows_v, sem).wait()
            pltpu.sync_copy(rows_v, out_hbm.at[pl.ds(off, chunk)])
            return carry

        lax.fori_loop(0, n_chunks, body, 0)

    return gather


def kernel(input_ids, emb_table, head_w, head_b):
    B, L = input_ids.shape
    V, D = emb_table.shape
    table = _make_table(emb_table, head_w, head_b)
    ids = input_ids.reshape(-1).astype(jnp.int32)
    N = B * L
    flat = _make_gather(N, V, n_workers=32, chunk=64)(table, ids)
    return flat.reshape(B, L, V)


# speedup vs baseline: 1.0242x; 1.0242x over previous
import functools

import jax
import jax.numpy as jnp
from jax import lax
from jax.experimental import pallas as pl
from jax.experimental.pallas import tpu as pltpu
from jax.experimental.pallas import tpu_sc as plsc


def _table_body(emb_ref, w_ref, b_ref, out_ref):
    acc = lax.dot_general(
        emb_ref[...], w_ref[...],
        dimension_numbers=(((1,), (1,)), ((), ())),
        preferred_element_type=jnp.float32,
    )
    out_ref[...] = acc + b_ref[...][None, :]


def _make_table(emb_table, head_w, head_b):
    V, D = emb_table.shape
    return pl.pallas_call(
        _table_body,
        out_shape=jax.ShapeDtypeStruct((V, V), jnp.float32),
    )(emb_table, head_w, head_b)


def _make_gather(N, V, n_workers, chunk):
    rows_per_w = N // n_workers
    n_chunks = rows_per_w // chunk
    mesh = plsc.VectorSubcoreMesh(core_axis_name="c", subcore_axis_name="s")

    @functools.partial(
        pl.kernel,
        mesh=mesh,
        out_type=jax.ShapeDtypeStruct((N, V), jnp.float32),
        scratch_types=[
            pltpu.VMEM((chunk,), jnp.int32),
            pltpu.VMEM((chunk,), jnp.int32),
            pltpu.VMEM((chunk, V), jnp.float32),
            pltpu.VMEM((chunk, V), jnp.float32),
            pltpu.SemaphoreType.DMA,
            pltpu.SemaphoreType.DMA,
            pltpu.SemaphoreType.DMA,
            pltpu.SemaphoreType.DMA,
        ],
        compiler_params=pltpu.CompilerParams(use_tc_tiling_on_sc=False),
    )
    def gather(table_hbm, idx_hbm, out_hbm, idx0, idx1, rows0, rows1,
               g0, g1, w0, w1):
        n_cores = lax.axis_size("c")
        wid = lax.axis_index("s") * n_cores + lax.axis_index("c")
        base = wid * rows_per_w
        idx = (idx0, idx1)
        rows = (rows0, rows1)
        gsem = (g0, g1)
        wsem = (w0, w1)

        gh = [None] * n_chunks
        wh = [None] * n_chunks

        def fire_gather(c):
            b = c & 1
            pltpu.sync_copy(idx_hbm.at[pl.ds(base + c * chunk, chunk)], idx[b])
            gh[c] = pltpu.async_copy(table_hbm.at[idx[b]], rows[b], gsem[b])

        fire_gather(0)
        for c in range(n_chunks):
            b = c & 1
            if c + 1 < n_chunks:
                if c >= 1:
                    wh[c - 1].wait()
                fire_gather(c + 1)
            gh[c].wait()
            wh[c] = pltpu.async_copy(
                rows[b], out_hbm.at[pl.ds(base + c * chunk, chunk)], wsem[b])
        if n_chunks >= 2:
            wh[n_chunks - 2].wait()
        wh[n_chunks - 1].wait()

    return gather


def kernel(input_ids, emb_table, head_w, head_b):
    B, L = input_ids.shape
    V, D = emb_table.shape
    table = _make_table(emb_table, head_w, head_b)
    ids = input_ids.reshape(-1).astype(jnp.int32)
    N = B * L
    flat = _make_gather(N, V, n_workers=32, chunk=64)(table, ids)
    return flat.reshape(B, L, V)


# speedup vs baseline: 1.1506x; 1.1234x over previous
import functools

import jax
import jax.numpy as jnp
from jax import lax
from jax.experimental import pallas as pl
from jax.experimental.pallas import tpu as pltpu
from jax.experimental.pallas import tpu_sc as plsc


def _make_x_gather(N, D, n_workers):
    rows_per_w = N // n_workers
    mesh = plsc.VectorSubcoreMesh(core_axis_name="c", subcore_axis_name="s")

    @functools.partial(
        pl.kernel,
        mesh=mesh,
        out_type=jax.ShapeDtypeStruct((N, D), jnp.float32),
        scratch_types=[
            pltpu.VMEM((rows_per_w,), jnp.int32),
            pltpu.VMEM((rows_per_w, D), jnp.float32),
            pltpu.SemaphoreType.DMA,
        ],
    )
    def gather(emb_hbm, idx_hbm, out_hbm, idx_v, rows_v, sem):
        n_cores = lax.axis_size("c")
        wid = lax.axis_index("s") * n_cores + lax.axis_index("c")
        base = wid * rows_per_w
        pltpu.sync_copy(idx_hbm.at[pl.ds(base, rows_per_w)], idx_v)
        pltpu.async_copy(emb_hbm.at[idx_v], rows_v, sem).wait()
        pltpu.sync_copy(rows_v, out_hbm.at[pl.ds(base, rows_per_w)])

    return gather


def _head_body(x_ref, w_ref, b_ref, out_ref):
    acc = lax.dot_general(
        x_ref[...], w_ref[...],
        dimension_numbers=(((1,), (1,)), ((), ())),
        preferred_element_type=jnp.float32,
    )
    acc = acc + b_ref[0][None, :]
    BB, L, V = out_ref.shape
    out_ref[...] = acc.reshape(BB, L, V)


def _make_head(B, L, D, V, bb):
    grid = (B // bb,)
    return pl.pallas_call(
        _head_body,
        grid=grid,
        in_specs=[
            pl.BlockSpec((bb * L, D), lambda i: (i, 0)),
            pl.BlockSpec((V, D), lambda i: (0, 0)),
            pl.BlockSpec((1, V), lambda i: (0, 0)),
        ],
        out_specs=pl.BlockSpec((bb, L, V), lambda i: (i, 0, 0)),
        out_shape=jax.ShapeDtypeStruct((B, L, V), jnp.float32),
    )


def kernel(input_ids, emb_table, head_w, head_b):
    B, L = input_ids.shape
    V, D = emb_table.shape
    ids = input_ids.reshape(-1).astype(jnp.int32)
    N = B * L
    x = _make_x_gather(N, D, n_workers=32)(emb_table, ids)
    return _make_head(B, L, D, V, bb=8)(x, head_w, head_b.reshape(1, V))


# speedup vs baseline: 3.7115x; 3.2256x over previous
import functools

import jax
import jax.numpy as jnp
from jax import lax
from jax.experimental import pallas as pl
from jax.experimental.pallas import tpu as pltpu
from jax.experimental.pallas import tpu_sc as plsc


def _make_x_gather(N, D, n_workers):
    rows_per_w = N // n_workers
    mesh = plsc.VectorSubcoreMesh(core_axis_name="c", subcore_axis_name="s")

    @functools.partial(
        pl.kernel,
        mesh=mesh,
        out_type=jax.ShapeDtypeStruct((N, D), jnp.float32),
        scratch_types=[
            pltpu.VMEM((rows_per_w,), jnp.int32),
            pltpu.VMEM((rows_per_w, D), jnp.float32),
            pltpu.SemaphoreType.DMA,
        ],
    )
    def gather(emb_hbm, idx_hbm, out_hbm, idx_v, rows_v, sem):
        n_cores = lax.axis_size("c")
        wid = lax.axis_index("s") * n_cores + lax.axis_index("c")
        base = wid * rows_per_w
        pltpu.sync_copy(idx_hbm.at[pl.ds(base, rows_per_w)], idx_v)
        pltpu.async_copy(emb_hbm.at[idx_v], rows_v, sem).wait()
        pltpu.sync_copy(rows_v, out_hbm.at[pl.ds(base, rows_per_w)])

    return gather


def _head_body(x_ref, w_ref, b_ref, out_ref):
    acc = lax.dot_general(
        w_ref[...], x_ref[...],
        dimension_numbers=(((1,), (1,)), ((), ())),
        preferred_element_type=jnp.float32,
    )
    out_ref[0] = acc + b_ref[...]


def _make_head(B, L, D, V):
    return pl.pallas_call(
        _head_body,
        grid=(L,),
        in_specs=[
            pl.BlockSpec((B, D), lambda l: (l, 0)),
            pl.BlockSpec((V, D), lambda l: (0, 0)),
            pl.BlockSpec((V, 1), lambda l: (0, 0)),
        ],
        out_specs=pl.BlockSpec((1, V, B), lambda l: (l, 0, 0)),
        out_shape=jax.ShapeDtypeStruct((L, V, B), jnp.float32),
    )


def kernel(input_ids, emb_table, head_w, head_b):
    B, L = input_ids.shape
    V, D = emb_table.shape
    ids = input_ids.T.reshape(-1).astype(jnp.int32)
    N = B * L
    x = _make_x_gather(N, D, n_workers=32)(emb_table, ids)
    out_phys = _make_head(B, L, D, V)(x, head_w, head_b.reshape(V, 1))
    return out_phys.transpose(2, 0, 1)
